# baseline (device time: 120982 ns/iter reference)
import jax
import jax.numpy as jnp
from jax import lax
from jax.experimental import pallas as pl
from jax.experimental.pallas import tpu as pltpu

N_DEV = 32
M_PER = 128
N_COLS = 64
M_TOT = N_DEV * M_PER


def _stage(v, d, k, desc=None):
    n, c = v.shape
    g = n // (2 * d)
    p = v.reshape(g, 2, d, c)
    a, b = p[:, 0], p[:, 1]
    lo = jnp.minimum(a, b)
    hi = jnp.maximum(a, b)
    kg = k // (2 * d)
    if k == n and desc is None:
        out = jnp.concatenate([lo[:, None], hi[:, None]], axis=1)
    else:
        gi = lax.broadcasted_iota(jnp.int32, (g, 1, 1), 0)
        up = ((gi // kg) & 1) == 0
        if k == n:
            up = up != desc
        newa = jnp.where(up, lo, hi)
        newb = jnp.where(up, hi, lo)
        out = jnp.concatenate([newa[:, None], newb[:, None]], axis=1)
    return out.reshape(n, c)


def _bitonic_sort(v, desc):
    n = v.shape[0]
    k = 2
    while k <= n:
        d = k // 2
        while d >= 1:
            v = _stage(v, d, k, desc)
            d //= 2
        k *= 2
    return v


def _merge_tree(v):
    n = v.shape[0]
    k = 2 * M_PER
    while k <= n:
        d = k // 2
        while d >= 1:
            v = _stage(v, d, k)
            d //= 2
        k *= 2
    return v


def _body(x_ref, out_ref, comm_ref, send_sems, recv_sems):
    my_pos = lax.axis_index("i")

    barrier_sem = pltpu.get_barrier_semaphore()
    for jj in range(1, N_DEV):
        peer = lax.rem(my_pos + jj, N_DEV)
        pl.semaphore_signal(
            barrier_sem, inc=1,
            device_id=(peer,), device_id_type=pl.DeviceIdType.MESH,
        )
    pl.semaphore_wait(barrier_sem, N_DEV - 1)

    desc = (my_pos & 1) == 1
    xs = _bitonic_sort(x_ref[:, :].astype(jnp.bfloat16), desc)
    comm_ref[pl.ds(my_pos, 1)] = xs[None]

    rdmas = []
    for jj in range(1, N_DEV):
        peer = lax.rem(my_pos + jj, N_DEV)
        rdma = pltpu.make_async_remote_copy(
            src_ref=comm_ref.at[my_pos],
            dst_ref=comm_ref.at[my_pos],
            send_sem=send_sems.at[jj - 1],
            recv_sem=recv_sems.at[jj - 1],
            device_id=(peer,),
            device_id_type=pl.DeviceIdType.MESH,
        )
        rdma.start()
        rdmas.append(rdma)
    for rdma in rdmas:
        rdma.wait()

    merged = _merge_tree(comm_ref[...].reshape(M_TOT, N_COLS))
    comm_ref[...] = merged.reshape(N_DEV, M_PER, N_COLS)
    out_ref[:, :] = comm_ref[pl.ds(my_pos, 1)][0]


def kernel(x):
    return pl.pallas_call(
        _body,
        out_shape=jax.ShapeDtypeStruct((M_PER, N_COLS), jnp.bfloat16),
        in_specs=[pl.BlockSpec(memory_space=pltpu.VMEM)],
        out_specs=pl.BlockSpec(memory_space=pltpu.VMEM),
        scratch_shapes=[
            pltpu.VMEM((N_DEV, M_PER, N_COLS), jnp.bfloat16),
            pltpu.SemaphoreType.DMA((N_DEV - 1,)),
            pltpu.SemaphoreType.DMA((N_DEV - 1,)),
        ],
        compiler_params=pltpu.CompilerParams(
            collective_id=0, vmem_limit_bytes=64 * 1024 * 1024
        ),
    )(x)


# device time: 40196 ns/iter; 3.0098x vs baseline; 3.0098x over previous
import jax
import jax.numpy as jnp
from jax import lax
from jax.experimental import pallas as pl
from jax.experimental.pallas import tpu as pltpu

N_DEV = 32
M_PER = 128
N_COLS = 64
M_TOT = N_DEV * M_PER


def _swap_pairs(v, d):
    n, c = v.shape
    p = v.reshape(n // (2 * d), 2, d, c)
    p = jnp.concatenate([p[:, 1:2], p[:, 0:1]], axis=1)
    return p.reshape(n, c)


def _bitonic_sort(v, desc):
    n = v.shape[0]
    row = lax.broadcasted_iota(jnp.int32, (n, 1), 0)
    k = 2
    while k <= n:
        d = k // 2
        while d >= 1:
            y = _swap_pairs(v, d)
            up = (row & k) == 0
            if k == n:
                up = up != desc
            lower = (row & d) == 0
            take_min = up == lower
            v = jnp.where(take_min, jnp.minimum(v, y), jnp.maximum(v, y))
            d //= 2
        k *= 2
    return v


def _merge_tree_packed(v):
    n, c = v.shape
    h = n // 2
    w = jnp.concatenate([v[:h], v[h:]], axis=1)
    row = lax.broadcasted_iota(jnp.int32, (h, 1), 0)
    lane = lax.broadcasted_iota(jnp.int32, (1, 2 * c), 1)
    k = 2 * M_PER
    while k <= h:
        d = k // 2
        while d >= 1:
            y = _swap_pairs(w, d)
            up = (lane < c) if k == h else ((row & k) == 0)
            lower = (row & d) == 0
            take_min = up == lower
            w = jnp.where(take_min, jnp.minimum(w, y), jnp.maximum(w, y))
            d //= 2
        k *= 2
    a, b = w[:, :c], w[:, c:]
    w = jnp.concatenate([jnp.minimum(a, b), jnp.maximum(a, b)], axis=1)
    d = h // 2
    while d >= 1:
        y = _swap_pairs(w, d)
        lower = (row & d) == 0
        w = jnp.where(lower, jnp.minimum(w, y), jnp.maximum(w, y))
        d //= 2
    return jnp.concatenate([w[:, :c], w[:, c:]], axis=0)


def _body(x_ref, out_ref, comm_ref, send_sems, recv_sems):
    my_pos = lax.axis_index("i")

    barrier_sem = pltpu.get_barrier_semaphore()
    for jj in range(1, N_DEV):
        peer = lax.rem(my_pos + jj, N_DEV)
        pl.semaphore_signal(
            barrier_sem, inc=1,
            device_id=(peer,), device_id_type=pl.DeviceIdType.MESH,
        )
    pl.semaphore_wait(barrier_sem, N_DEV - 1)

    desc = (my_pos & 1) == 1
    xs = _bitonic_sort(x_ref[:, :].astype(jnp.bfloat16), desc)
    comm_ref[pl.ds(my_pos, 1)] = xs[None]

    rdmas = []
    for jj in range(1, N_DEV):
        peer = lax.rem(my_pos + jj, N_DEV)
        rdma = pltpu.make_async_remote_copy(
            src_ref=comm_ref.at[my_pos],
            dst_ref=comm_ref.at[my_pos],
            send_sem=send_sems.at[jj - 1],
            recv_sem=recv_sems.at[jj - 1],
            device_id=(peer,),
            device_id_type=pl.DeviceIdType.MESH,
        )
        rdma.start()
        rdmas.append(rdma)
    for rdma in rdmas:
        rdma.wait()

    merged = _merge_tree_packed(comm_ref[...].reshape(M_TOT, N_COLS))
    comm_ref[...] = merged.reshape(N_DEV, M_PER, N_COLS)
    out_ref[:, :] = comm_ref[pl.ds(my_pos, 1)][0]


def kernel(x):
    return pl.pallas_call(
        _body,
        out_shape=jax.ShapeDtypeStruct((M_PER, N_COLS), jnp.bfloat16),
        in_specs=[pl.BlockSpec(memory_space=pltpu.VMEM)],
        out_specs=pl.BlockSpec(memory_space=pltpu.VMEM),
        scratch_shapes=[
            pltpu.VMEM((N_DEV, M_PER, N_COLS), jnp.bfloat16),
            pltpu.SemaphoreType.DMA((N_DEV - 1,)),
            pltpu.SemaphoreType.DMA((N_DEV - 1,)),
        ],
        compiler_params=pltpu.CompilerParams(
            collective_id=0, vmem_limit_bytes=64 * 1024 * 1024
        ),
    )(x)


# device time: 30851 ns/iter; 3.9215x vs baseline; 1.3029x over previous
import jax
import jax.numpy as jnp
from jax import lax
from jax.experimental import pallas as pl
from jax.experimental.pallas import tpu as pltpu

N_DEV = 32
M_PER = 128
N_COLS = 64
M_TOT = N_DEV * M_PER
G_SIZE = 8
N_GRP = N_DEV // G_SIZE
M_GRP = G_SIZE * M_PER


def _swap_pairs(v, d):
    n, c = v.shape
    p = v.reshape(n // (2 * d), 2, d, c)
    p = jnp.concatenate([p[:, 1:2], p[:, 0:1]], axis=1)
    return p.reshape(n, c)


def _bitonic_sort(v, desc):
    n = v.shape[0]
    row = lax.broadcasted_iota(jnp.int32, (n, 1), 0)
    k = 2
    while k <= n:
        d = k // 2
        while d >= 1:
            y = _swap_pairs(v, d)
            up = (row & k) == 0
            if k == n:
                up = up != desc
            lower = (row & d) == 0
            take_min = up == lower
            v = jnp.where(take_min, jnp.minimum(v, y), jnp.maximum(v, y))
            d //= 2
        k *= 2
    return v


def _merge_packed(v, first_k, desc=None):
    n, c = v.shape
    h = n // 2
    w = jnp.concatenate([v[:h], v[h:]], axis=1)
    row = lax.broadcasted_iota(jnp.int32, (h, 1), 0)
    lane = lax.broadcasted_iota(jnp.int32, (1, 2 * c), 1)
    k = first_k
    while k <= h:
        d = k // 2
        while d >= 1:
            y = _swap_pairs(w, d)
            up = (lane < c) if k == h else ((row & k) == 0)
            lower = (row & d) == 0
            take_min = up == lower
            w = jnp.where(take_min, jnp.minimum(w, y), jnp.maximum(w, y))
            d //= 2
        k *= 2
    a, b = w[:, :c], w[:, c:]
    lo, hi = jnp.minimum(a, b), jnp.maximum(a, b)
    if desc is None:
        w = jnp.concatenate([lo, hi], axis=1)
    else:
        wa = jnp.where(desc, hi, lo)
        wb = jnp.where(desc, lo, hi)
        w = jnp.concatenate([wa, wb], axis=1)
    d = h // 2
    while d >= 1:
        y = _swap_pairs(w, d)
        lower = (row & d) == 0
        take_min = lower if desc is None else (lower != desc)
        w = jnp.where(take_min, jnp.minimum(w, y), jnp.maximum(w, y))
        d //= 2
    return jnp.concatenate([w[:, :c], w[:, c:]], axis=0)


def _body(x_ref, out_ref, comm_ref, send_sems, recv_sems):
    my_pos = lax.axis_index("i")
    my_grp = my_pos // G_SIZE
    my_rank = lax.rem(my_pos, G_SIZE)
    grp_base = my_grp * G_SIZE

    barrier_sem = pltpu.get_barrier_semaphore()
    n_peers = (G_SIZE - 1) + (N_GRP - 1)
    for jj in range(1, G_SIZE):
        peer = grp_base + lax.rem(my_rank + jj, G_SIZE)
        pl.semaphore_signal(
            barrier_sem, inc=1,
            device_id=(peer,), device_id_type=pl.DeviceIdType.MESH,
        )
    for gg in range(1, N_GRP):
        peer = lax.rem(my_grp + gg, N_GRP) * G_SIZE + my_rank
        pl.semaphore_signal(
            barrier_sem, inc=1,
            device_id=(peer,), device_id_type=pl.DeviceIdType.MESH,
        )
    pl.semaphore_wait(barrier_sem, n_peers)

    desc = (my_pos & 1) == 1
    xs = _bitonic_sort(x_ref[:, :].astype(jnp.bfloat16), desc)
    comm_ref[pl.ds(my_pos, 1)] = xs[None]

    l1 = []
    for jj in range(1, G_SIZE):
        peer = grp_base + lax.rem(my_rank + jj, G_SIZE)
        rdma = pltpu.make_async_remote_copy(
            src_ref=comm_ref.at[my_pos],
            dst_ref=comm_ref.at[my_pos],
            send_sem=send_sems.at[jj - 1],
            recv_sem=recv_sems.at[jj - 1],
            device_id=(peer,),
            device_id_type=pl.DeviceIdType.MESH,
        )
        rdma.start()
        l1.append(rdma)
    for rdma in l1:
        rdma.wait()

    gdesc = (my_grp & 1) == 1
    grun = _merge_packed(
        comm_ref[pl.ds(grp_base, G_SIZE)].reshape(M_GRP, N_COLS),
        2 * M_PER,
        gdesc,
    )
    comm_ref[pl.ds(grp_base, G_SIZE)] = grun.reshape(G_SIZE, M_PER, N_COLS)

    l2 = []
    for gg in range(1, N_GRP):
        peer = lax.rem(my_grp + gg, N_GRP) * G_SIZE + my_rank
        rdma = pltpu.make_async_remote_copy(
            src_ref=comm_ref.at[pl.ds(grp_base, G_SIZE)],
            dst_ref=comm_ref.at[pl.ds(grp_base, G_SIZE)],
            send_sem=send_sems.at[G_SIZE - 2 + gg],
            recv_sem=recv_sems.at[G_SIZE - 2 + gg],
            device_id=(peer,),
            device_id_type=pl.DeviceIdType.MESH,
        )
        rdma.start()
        l2.append(rdma)
    for rdma in l2:
        rdma.wait()

    merged = _merge_packed(comm_ref[...].reshape(M_TOT, N_COLS), 2 * M_GRP)
    comm_ref[...] = merged.reshape(N_DEV, M_PER, N_COLS)
    out_ref[:, :] = comm_ref[pl.ds(my_pos, 1)][0]


def kernel(x):
    return pl.pallas_call(
        _body,
        out_shape=jax.ShapeDtypeStruct((M_PER, N_COLS), jnp.bfloat16),
        in_specs=[pl.BlockSpec(memory_space=pltpu.VMEM)],
        out_specs=pl.BlockSpec(memory_space=pltpu.VMEM),
        scratch_shapes=[
            pltpu.VMEM((N_DEV, M_PER, N_COLS), jnp.bfloat16),
            pltpu.SemaphoreType.DMA((G_SIZE - 1 + N_GRP - 1,)),
            pltpu.SemaphoreType.DMA((G_SIZE - 1 + N_GRP - 1,)),
        ],
        compiler_params=pltpu.CompilerParams(
            collective_id=0, vmem_limit_bytes=64 * 1024 * 1024
        ),
    )(x)
